# Initial kernel scaffold; baseline (speedup 1.0000x reference)
import jax
import jax.numpy as jnp
from jax.experimental import pallas as pl


def kernel(features, edge_index, W_self1, W_neigh1, b1, W_self2, W_neigh2, b2):
    raise NotImplementedError("write your pallas kernel here")


# speedup vs baseline: 6.2099x; 6.2099x over previous
import functools

import jax
import jax.numpy as jnp
from jax import lax
from jax.experimental import pallas as pl
from jax.experimental.pallas import tpu as pltpu
from jax.experimental.pallas import tpu_sc as plsc

N_NODES = 10000
N_PAD = 10240
E = 320000
D_IN = 128
D_TAB = 144
NCLS = 47

NC = 2
NS = 16
NW = NC * NS
EPW = E // NW
B = 80
NCHUNK = EPW // B
ROWS_PT = N_PAD // NS


def _make_agg(D):
    mesh = plsc.VectorSubcoreMesh(core_axis_name="c", subcore_axis_name="s")

    @functools.partial(
        pl.kernel,
        mesh=mesh,
        compiler_params=pltpu.CompilerParams(use_tc_tiling_on_sc=False),
        out_type=jax.ShapeDtypeStruct((NC, N_PAD, D), jnp.float32),
        scratch_types=[
            pltpu.VMEM((B,), jnp.int32),
            pltpu.VMEM((B,), jnp.int32),
            pltpu.VMEM((B, D), jnp.float32),
            pltpu.VMEM_SHARED((N_PAD, D), jnp.float32),
            pltpu.SemaphoreType.DMA,
        ],
    )
    def agg(table_hbm, src_hbm, dst_hbm, out_hbm, src_v, dst_v, rows_v, acc_sh, sem):
        c = lax.axis_index("c")
        s = lax.axis_index("s")
        wid = s * NC + c

        def zrow(i, carry):
            for b in range(D // 16):
                rows_v[i, pl.ds(b * 16, 16)] = jnp.zeros((16,), jnp.float32)
            return carry

        lax.fori_loop(0, B, zrow, 0)
        for j in range(ROWS_PT // B):
            pltpu.sync_copy(rows_v, acc_sh.at[pl.ds(s * ROWS_PT + j * B, B)])
        plsc.subcore_barrier()

        base0 = wid * EPW

        def body(i, carry):
            base = base0 + i * B
            pltpu.sync_copy(src_hbm.at[pl.ds(base, B)], src_v)
            pltpu.sync_copy(dst_hbm.at[pl.ds(base, B)], dst_v)
            pltpu.async_copy(table_hbm.at[src_v], rows_v, sem).wait()
            pltpu.sync_copy(rows_v, acc_sh.at[dst_v], add=True)
            return carry

        lax.fori_loop(0, NCHUNK, body, 0)
        plsc.subcore_barrier()
        pltpu.sync_copy(
            acc_sh.at[pl.ds(s * ROWS_PT, ROWS_PT)],
            out_hbm.at[c, pl.ds(s * ROWS_PT, ROWS_PT)],
        )

    return agg


_agg_tab = _make_agg(D_TAB)
_agg_feat = _make_agg(D_IN)


def _combine_body(p_ref, m_ref, d_ref):
    s = p_ref[0] + p_ref[1]
    deg = s[:, 128:129]
    m_ref[...] = s[:, :128] / jnp.maximum(deg, 1.0)
    d_ref[...] = deg


_R1 = 2048


def _combine(p1):
    return pl.pallas_call(
        _combine_body,
        grid=(N_PAD // _R1,),
        in_specs=[pl.BlockSpec((NC, _R1, D_TAB), lambda i: (0, i, 0))],
        out_specs=[
            pl.BlockSpec((_R1, D_IN), lambda i: (i, 0)),
            pl.BlockSpec((_R1, 1), lambda i: (i, 0)),
        ],
        out_shape=[
            jax.ShapeDtypeStruct((N_PAD, D_IN), jnp.float32),
            jax.ShapeDtypeStruct((N_PAD, 1), jnp.float32),
        ],
    )(p1)


def _wcomb_body(ws1, wn1, ws2, wn2, b1, b2, wa, wb, wc, cm):
    f32 = jnp.float32
    wa[...] = jnp.dot(ws1[...], ws2[...], preferred_element_type=f32)
    wb[...] = jnp.dot(wn1[...], ws2[...], preferred_element_type=f32) + jnp.dot(
        ws1[...], wn2[...], preferred_element_type=f32
    )
    wc[...] = jnp.dot(wn1[...], wn2[...], preferred_element_type=f32)
    cm[0:1, :] = jnp.dot(b1[...], ws2[...], preferred_element_type=f32) + b2[...]
    cm[1:2, :] = jnp.dot(b1[...], wn2[...], preferred_element_type=f32)


def _wcomb(Ws1, Wn1, Ws2, Wn2, b1, b2):
    sh = jax.ShapeDtypeStruct
    return pl.pallas_call(
        _wcomb_body,
        out_shape=[
            sh((D_IN, NCLS), jnp.float32),
            sh((D_IN, NCLS), jnp.float32),
            sh((D_IN, NCLS), jnp.float32),
            sh((2, NCLS), jnp.float32),
        ],
    )(Ws1, Wn1, Ws2, Wn2, b1, b2)


_R2 = 2000


def _final_body(f_ref, m_ref, p2_ref, d_ref, wa_ref, wb_ref, wc_ref, cm_ref, o_ref):
    deg = d_ref[...]
    dmax = jnp.maximum(deg, 1.0)
    m1m = (p2_ref[0] + p2_ref[1]) / dmax
    r = (deg > 0.0).astype(jnp.float32)
    f32 = jnp.float32
    acc = jnp.dot(f_ref[...], wa_ref[...], preferred_element_type=f32)
    acc += jnp.dot(m_ref[...], wb_ref[...], preferred_element_type=f32)
    acc += jnp.dot(m1m, wc_ref[...], preferred_element_type=f32)
    acc += cm_ref[0:1, :] + r * cm_ref[1:2, :]
    o_ref[...] = acc


def _final(features, m1tab, p2, deg, wa, wb, wc, cm):
    return pl.pallas_call(
        _final_body,
        grid=(N_NODES // _R2,),
        in_specs=[
            pl.BlockSpec((_R2, D_IN), lambda i: (i, 0)),
            pl.BlockSpec((_R2, D_IN), lambda i: (i, 0)),
            pl.BlockSpec((NC, _R2, D_IN), lambda i: (0, i, 0)),
            pl.BlockSpec((_R2, 1), lambda i: (i, 0)),
            pl.BlockSpec((D_IN, NCLS), lambda i: (0, 0)),
            pl.BlockSpec((D_IN, NCLS), lambda i: (0, 0)),
            pl.BlockSpec((D_IN, NCLS), lambda i: (0, 0)),
            pl.BlockSpec((2, NCLS), lambda i: (0, 0)),
        ],
        out_specs=pl.BlockSpec((_R2, NCLS), lambda i: (i, 0)),
        out_shape=jax.ShapeDtypeStruct((N_NODES, NCLS), jnp.float32),
    )(features, m1tab, p2, deg, wa, wb, wc, cm)


def kernel(features, edge_index, W_self1, W_neigh1, b1, W_self2, W_neigh2, b2):
    src = edge_index[0].astype(jnp.int32)
    dst = edge_index[1].astype(jnp.int32)

    ones = jnp.ones((N_NODES, 1), jnp.float32)
    zpad = jnp.zeros((N_NODES, D_TAB - D_IN - 1), jnp.float32)
    ftab = jnp.concatenate([features, ones, zpad], axis=1)
    ftab = jnp.pad(ftab, ((0, N_PAD - N_NODES), (0, 0)))

    p1 = _agg_tab(ftab, src, dst)
    m1tab, deg = _combine(p1)
    p2 = _agg_feat(m1tab, src, dst)
    wa, wb, wc, cm = _wcomb(
        W_self1, W_neigh1, W_self2, W_neigh2,
        b1.reshape(1, -1), b2.reshape(1, -1),
    )
    return _final(features, m1tab, p2, deg, wa, wb, wc, cm)
